# Initial kernel scaffold; baseline (speedup 1.0000x reference)
import jax
import jax.numpy as jnp
from jax.experimental import pallas as pl


def kernel(inputs, embedding):
    raise NotImplementedError("write your pallas kernel here")


# speedup vs baseline: 1.2070x; 1.2070x over previous
import functools

import jax
import jax.numpy as jnp
from jax import lax
from jax.experimental import pallas as pl
from jax.experimental.pallas import tpu as pltpu
from jax.experimental.pallas import tpu_sc as plsc

_NUM_EMB = 8192
_DIM = 32
_COMMIT = 0.25
_T = 256


def _dist_argmin_body(x_ref, et_ref, esq_ref, idx_ref, loss_ref):
    x = x_ref[...]
    mm = lax.dot_general(
        x.astype(jnp.bfloat16), et_ref[...].astype(jnp.bfloat16),
        (((1,), (0,)), ((), ())),
        preferred_element_type=jnp.float32,
    )
    xsq = jnp.sum(x ** 2, axis=1, keepdims=True)
    d = (xsq + esq_ref[...]) - 2.0 * mm
    idx_ref[...] = jnp.argmin(d, axis=1)[:, None]

    @pl.when(pl.program_id(0) == 0)
    def _init():
        loss_ref[...] = jnp.zeros((1, 1), jnp.float32)

    loss_ref[...] += jnp.sum(jnp.min(d, axis=1)).reshape(1, 1)


def _dist_argmin(flat_x, et, esq):
    n = flat_x.shape[0]
    return pl.pallas_call(
        _dist_argmin_body,
        grid=(n // _T,),
        in_specs=[
            pl.BlockSpec((_T, _DIM), lambda i: (i, 0)),
            pl.BlockSpec((_DIM, _NUM_EMB), lambda i: (0, 0)),
            pl.BlockSpec((1, _NUM_EMB), lambda i: (0, 0)),
        ],
        out_specs=[
            pl.BlockSpec((_T, 1), lambda i: (i, 0)),
            pl.BlockSpec((1, 1), lambda i: (0, 0)),
        ],
        out_shape=[
            jax.ShapeDtypeStruct((n, 1), jnp.int32),
            jax.ShapeDtypeStruct((1, 1), jnp.float32),
        ],
    )(flat_x, et, esq)


def _make_sc_gather(n_tokens):
    info = plsc.get_sparse_core_info()
    nw = info.num_cores * info.num_subcores
    b_per_w = n_tokens // nw
    mesh = plsc.VectorSubcoreMesh(core_axis_name="c", subcore_axis_name="s")

    @functools.partial(
        pl.kernel,
        mesh=mesh,
        compiler_params=pltpu.CompilerParams(use_tc_tiling_on_sc=False),
        out_type=jax.ShapeDtypeStruct((n_tokens, _DIM), jnp.float32),
        scratch_types=[
            pltpu.VMEM((b_per_w,), jnp.int32),
            pltpu.VMEM((b_per_w, _DIM), jnp.float32),
            pltpu.SemaphoreType.DMA,
        ],
    )
    def _gather(table_hbm, idx_hbm, out_hbm, idx_v, rows_v, sem):
        wid = lax.axis_index("s") * info.num_cores + lax.axis_index("c")
        base = wid * b_per_w
        pltpu.sync_copy(idx_hbm.at[pl.ds(base, b_per_w)], idx_v)
        pltpu.async_copy(table_hbm.at[idx_v], rows_v, sem).wait()
        pltpu.sync_copy(rows_v, out_hbm.at[pl.ds(base, b_per_w)])

    return _gather


def kernel(inputs, embedding):
    input_dtype = inputs.dtype
    x = inputs.astype(jnp.float32)
    flat = x.reshape(-1, _DIM)
    n = flat.shape[0]
    et = embedding.T
    esq = jnp.sum(embedding ** 2, axis=1)[None, :]
    idx2d, loss_sum = _dist_argmin(flat, et, esq)
    quantized = _make_sc_gather(n)(embedding, idx2d.reshape(-1))
    quantized = quantized.reshape(x.shape).astype(input_dtype)
    loss = _COMMIT * (loss_sum[0, 0] / (n * _DIM))
    return (quantized, loss, idx2d)
